# baseline (speedup 1.0000x reference)
import jax
import jax.numpy as jnp
from jax import lax
from jax.experimental import pallas as pl
from jax.experimental.pallas import tpu as pltpu
from jax.experimental.pallas import tpu_sc as plsc

NV = 10000
NE = 2500
NI = 320000
D = 128
NC = 2
NS = 16
BLK = 128
CH = 16
NBLK_T = 80
NI_PAD = NC * NS * NBLK_T * BLK
E_TAB = 2560
V_TAB = 10112
E_ROWS_T = E_TAB // NS
V_ROWS_T = V_TAB // NS

_f32 = jnp.float32


def _make_stage(tab_rows, rows_per_tile, nbuf):
    def body(src_hbm, gidx_hbm, sidx_hbm, z_hbm, out_hbm, gi, si, *rest):
        rows = rest[:nbuf]
        gsem = rest[nbuf:2 * nbuf]
        ssem = rest[2 * nbuf:3 * nbuf]
        tab = rest[3 * nbuf]
        c = lax.axis_index("c")
        s = lax.axis_index("s")
        pltpu.sync_copy(gidx_hbm.at[c, s], gi)
        pltpu.sync_copy(sidx_hbm.at[c, s], si)
        pltpu.sync_copy(z_hbm.at[pl.ds(s * rows_per_tile, rows_per_tile)],
                        tab.at[pl.ds(s * rows_per_tile, rows_per_tile)])
        plsc.subcore_barrier()

        if nbuf == 1:
            def group(j, _):
                pltpu.async_copy(src_hbm.at[gi.at[j]], rows[0],
                                 gsem[0]).wait()
                pltpu.sync_copy(rows[0], tab.at[si.at[j]], add=True)
                return 0

            lax.fori_loop(0, NBLK_T, group, 0)
        else:
            def group(g, _):
                g0 = g * nbuf
                gd = [pltpu.async_copy(src_hbm.at[gi.at[g0 + b]], rows[b],
                                       gsem[b]) for b in range(nbuf)]
                for b in range(nbuf):
                    gd[b].wait()
                    pltpu.async_copy(rows[b], tab.at[si.at[g0 + b]],
                                     ssem[b], add=True)
                for b in range(nbuf):
                    pltpu.make_async_copy(rows[b], tab.at[si.at[g0 + b]],
                                          ssem[b]).wait()
                return 0

            lax.fori_loop(0, NBLK_T // nbuf, group, 0)
        plsc.subcore_barrier()
        pltpu.sync_copy(tab.at[pl.ds(s * rows_per_tile, rows_per_tile)],
                        out_hbm.at[c, pl.ds(s * rows_per_tile, rows_per_tile)])

    return pl.kernel(
        body,
        out_type=jax.ShapeDtypeStruct((NC, tab_rows, D), _f32),
        mesh=plsc.VectorSubcoreMesh(core_axis_name="c", subcore_axis_name="s",
                                    num_cores=NC, num_subcores=NS),
        scratch_types=(
            [pltpu.VMEM((NBLK_T, BLK), jnp.int32)] * 2
            + [pltpu.VMEM((BLK, D), _f32)] * nbuf
            + [pltpu.SemaphoreType.DMA] * (2 * nbuf)
            + [pltpu.VMEM_SHARED((tab_rows, D), _f32)]
        ),
    )


_v2e = _make_stage(E_TAB, E_ROWS_T, 1)
_e2v = _make_stage(V_TAB, V_ROWS_T, 1)


def _prep_body(pd_ref, wt_ref, bt_ref, wl0, bl0, wl1, bl1, wl2, bl2, topos):
    h = jnp.dot(pd_ref[...], wt_ref[...], preferred_element_type=_f32)
    h = jnp.maximum(h + bt_ref[...], 0.0)
    t = jnp.mean(h, axis=0, keepdims=True)
    t1 = jnp.dot(t, wl0[...], preferred_element_type=_f32) + bl0[...]
    t2 = jnp.dot(t1, wl1[...], preferred_element_type=_f32) + bl1[...]
    t3 = jnp.dot(t2, wl2[...], preferred_element_type=_f32) + bl2[...]
    topos[0:1, :] = t1
    topos[1:2, :] = t2
    topos[2:3, :] = t3


_prep = pl.pallas_call(
    _prep_body, out_shape=jax.ShapeDtypeStruct((3, D), _f32))


def _mm0_body(x_ref, w_ref, b_ref, o_ref):
    o_ref[...] = jnp.dot(x_ref[...], w_ref[...],
                         preferred_element_type=_f32) + b_ref[...]


_mm0 = pl.pallas_call(
    _mm0_body, out_shape=jax.ShapeDtypeStruct((NV, D), _f32))


def _combe_body(ep_ref, ec_ref, o_ref):
    p = ep_ref[0] + ep_ref[1]
    deg = ec_ref[0, :, 0:1] + ec_ref[1, :, 0:1]
    r = p / jnp.clip(deg, 1.0, None)
    o_ref[...] = r[:NE]


_combe = pl.pallas_call(
    _combe_body, out_shape=jax.ShapeDtypeStruct((NE, D), _f32))


def _combv(qp_ref, vc_ref, topo_ref):
    q = qp_ref[0] + qp_ref[1]
    deg = vc_ref[0, :, 0:1] + vc_ref[1, :, 0:1]
    xc = q / jnp.clip(deg, 1.0, None)
    return jnp.maximum(xc * (1.0 + topo_ref[...]), 0.0)


def _cvm_body(qp_ref, vc_ref, topo_ref, w_ref, b_ref, o_ref):
    xa = _combv(qp_ref, vc_ref, topo_ref)
    o_ref[...] = jnp.dot(xa[:NV], w_ref[...],
                         preferred_element_type=_f32) + b_ref[...]


_cvm = pl.pallas_call(
    _cvm_body, out_shape=jax.ShapeDtypeStruct((NV, D), _f32))


def _final_body(qp_ref, vc_ref, topo_ref, o_ref):
    o_ref[...] = _combv(qp_ref, vc_ref, topo_ref)[:NV]


_final = pl.pallas_call(
    _final_body, out_shape=jax.ShapeDtypeStruct((NV, D), _f32))


def kernel(x, pd, hg_vertex_index, hg_hyperedge_index, W_topo, b_topo,
           W_g0, b_g0, W_l0, b_l0, W_g1, b_g1, W_l1, b_l1,
           W_g2, b_g2, W_l2, b_l2):
    vi = hg_vertex_index.astype(jnp.int32)
    ei = hg_hyperedge_index.astype(jnp.int32)
    pad = NI_PAD - NI
    shape4 = (NC, NS, NBLK_T, BLK)
    vg = jnp.concatenate([vi, jnp.zeros((pad,), jnp.int32)]).reshape(shape4)
    eg = jnp.concatenate([ei, jnp.zeros((pad,), jnp.int32)]).reshape(shape4)
    es = jnp.concatenate([ei, jnp.full((pad,), NE, jnp.int32)]).reshape(shape4)
    vs = jnp.concatenate([vi, jnp.full((pad,), NV, jnp.int32)]).reshape(shape4)

    pd_p = jnp.pad(pd, ((0, 0), (0, 3)))
    wt_p = jnp.pad(W_topo, ((0, 3), (0, 0)))

    topos = _prep(pd_p, wt_p, b_topo, W_l0, b_l0, W_l1, b_l1, W_l2, b_l2)

    ze = jnp.zeros((E_TAB, D), _f32)
    zv = jnp.zeros((V_TAB, D), _f32)
    ecnt = _v2e(jnp.ones((NV, D), _f32), vg, es, ze)
    vcnt = _e2v(jnp.ones((NE, D), _f32), eg, vs, zv)

    Wg = [(W_g0, b_g0), (W_g1, b_g1), (W_g2, b_g2)]
    X = _mm0(x, W_g0, b_g0)
    for i in range(3):
        ep = _v2e(X, vg, es, ze)
        ef = _combe(ep, ecnt)
        qp = _e2v(ef, eg, vs, zv)
        if i < 2:
            X = _cvm(qp, vcnt, topos[i:i + 1], Wg[i + 1][0], Wg[i + 1][1])
        else:
            xf = _final(qp, vcnt, topos[2:3])
    return (xf, topos[2:3])


# speedup vs baseline: 1.5942x; 1.5942x over previous
import jax
import jax.numpy as jnp
from jax import lax
from jax.experimental import pallas as pl
from jax.experimental.pallas import tpu as pltpu
from jax.experimental.pallas import tpu_sc as plsc

NV = 10000
NE = 2500
NI = 320000
D = 128
NC = 2
NS = 16
BLK = 128
CH = 16
NBLK_T = 79
NI_PAD = NC * NS * NBLK_T * BLK
E_TAB = 2560
V_TAB = 10112
E_ROWS_T = E_TAB // NS
V_ROWS_T = V_TAB // NS

_f32 = jnp.float32


def _make_stage(tab_rows, rows_per_tile, nbuf):
    def body(src_hbm, gidx_hbm, sidx_hbm, z_hbm, out_hbm, gi, si, *rest):
        rows = rest[:nbuf]
        nsem = 2 * nbuf if nbuf > 1 else 1
        gsem = rest[nbuf:nbuf + nsem]
        ssem = gsem[nbuf:]
        tab = rest[nbuf + nsem]
        c = lax.axis_index("c")
        s = lax.axis_index("s")
        pltpu.sync_copy(gidx_hbm.at[c, s], gi)
        pltpu.sync_copy(sidx_hbm.at[c, s], si)
        pltpu.sync_copy(z_hbm.at[pl.ds(s * rows_per_tile, rows_per_tile)],
                        tab.at[pl.ds(s * rows_per_tile, rows_per_tile)])
        plsc.subcore_barrier()

        if nbuf == 1:
            def group(j, _):
                pltpu.async_copy(src_hbm.at[gi.at[j]], rows[0],
                                 gsem[0]).wait()
                pltpu.sync_copy(rows[0], tab.at[si.at[j]], add=True)
                return 0

            lax.fori_loop(0, NBLK_T, group, 0)
        else:
            def group(g, _):
                g0 = g * nbuf
                gd = [pltpu.async_copy(src_hbm.at[gi.at[g0 + b]], rows[b],
                                       gsem[b]) for b in range(nbuf)]
                for b in range(nbuf):
                    gd[b].wait()
                    pltpu.async_copy(rows[b], tab.at[si.at[g0 + b]],
                                     ssem[b], add=True)
                for b in range(nbuf):
                    pltpu.make_async_copy(rows[b], tab.at[si.at[g0 + b]],
                                          ssem[b]).wait()
                return 0

            lax.fori_loop(0, NBLK_T // nbuf, group, 0)
        plsc.subcore_barrier()
        pltpu.sync_copy(tab.at[pl.ds(s * rows_per_tile, rows_per_tile)],
                        out_hbm.at[c, pl.ds(s * rows_per_tile, rows_per_tile)])

    return pl.kernel(
        body,
        out_type=jax.ShapeDtypeStruct((NC, tab_rows, D), _f32),
        mesh=plsc.VectorSubcoreMesh(core_axis_name="c", subcore_axis_name="s",
                                    num_cores=NC, num_subcores=NS),
        scratch_types=(
            [pltpu.VMEM((NBLK_T, BLK), jnp.int32)] * 2
            + [pltpu.VMEM((BLK, D), _f32)] * nbuf
            + [pltpu.SemaphoreType.DMA] * (2 * nbuf if nbuf > 1 else 1)
            + [pltpu.VMEM_SHARED((tab_rows, D), _f32)]
        ),
    )


_v2e = _make_stage(E_TAB, E_ROWS_T, 1)
_e2v = _make_stage(V_TAB, V_ROWS_T, 1)


def _prep_body(pd_ref, wt_ref, bt_ref, wl0, bl0, wl1, bl1, wl2, bl2, topos):
    h = jnp.dot(pd_ref[...], wt_ref[...], preferred_element_type=_f32)
    h = jnp.maximum(h + bt_ref[...], 0.0)
    t = jnp.mean(h, axis=0, keepdims=True)
    t1 = jnp.dot(t, wl0[...], preferred_element_type=_f32) + bl0[...]
    t2 = jnp.dot(t1, wl1[...], preferred_element_type=_f32) + bl1[...]
    t3 = jnp.dot(t2, wl2[...], preferred_element_type=_f32) + bl2[...]
    topos[0:1, :] = t1
    topos[1:2, :] = t2
    topos[2:3, :] = t3


_prep = pl.pallas_call(
    _prep_body, out_shape=jax.ShapeDtypeStruct((3, D), _f32))


def _mm0_body(x_ref, w_ref, b_ref, o_ref):
    o_ref[...] = jnp.dot(x_ref[...], w_ref[...],
                         preferred_element_type=_f32) + b_ref[...]


_mm0 = pl.pallas_call(
    _mm0_body, out_shape=jax.ShapeDtypeStruct((NV, D), _f32))


def _combe_body(ep_ref, ec_ref, o_ref):
    p = ep_ref[0] + ep_ref[1]
    deg = ec_ref[0, :, 0:1] + ec_ref[1, :, 0:1]
    r = p / jnp.clip(deg, 1.0, None)
    o_ref[...] = r[:NE]


_combe = pl.pallas_call(
    _combe_body, out_shape=jax.ShapeDtypeStruct((NE, D), _f32))


def _combv(qp_ref, vc_ref, topo_ref):
    q = qp_ref[0] + qp_ref[1]
    deg = vc_ref[0, :, 0:1] + vc_ref[1, :, 0:1]
    xc = q / jnp.clip(deg, 1.0, None)
    return jnp.maximum(xc * (1.0 + topo_ref[...]), 0.0)


def _cvm_body(qp_ref, vc_ref, topo_ref, w_ref, b_ref, o_ref):
    xa = _combv(qp_ref, vc_ref, topo_ref)
    o_ref[...] = jnp.dot(xa[:NV], w_ref[...],
                         preferred_element_type=_f32) + b_ref[...]


_cvm = pl.pallas_call(
    _cvm_body, out_shape=jax.ShapeDtypeStruct((NV, D), _f32))


def _final_body(qp_ref, vc_ref, topo_ref, o_ref):
    o_ref[...] = _combv(qp_ref, vc_ref, topo_ref)[:NV]


_final = pl.pallas_call(
    _final_body, out_shape=jax.ShapeDtypeStruct((NV, D), _f32))


def kernel(x, pd, hg_vertex_index, hg_hyperedge_index, W_topo, b_topo,
           W_g0, b_g0, W_l0, b_l0, W_g1, b_g1, W_l1, b_l1,
           W_g2, b_g2, W_l2, b_l2):
    vi = hg_vertex_index.astype(jnp.int32)
    ei = hg_hyperedge_index.astype(jnp.int32)
    pad = NI_PAD - NI
    shape4 = (NC, NS, NBLK_T, BLK)
    vg = jnp.concatenate([vi, jnp.zeros((pad,), jnp.int32)]).reshape(shape4)
    eg = jnp.concatenate([ei, jnp.zeros((pad,), jnp.int32)]).reshape(shape4)
    es = jnp.concatenate([ei, jnp.full((pad,), NE, jnp.int32)]).reshape(shape4)
    vs = jnp.concatenate([vi, jnp.full((pad,), NV, jnp.int32)]).reshape(shape4)

    pd_p = jnp.pad(pd, ((0, 0), (0, 3)))
    wt_p = jnp.pad(W_topo, ((0, 3), (0, 0)))

    topos = _prep(pd_p, wt_p, b_topo, W_l0, b_l0, W_l1, b_l1, W_l2, b_l2)

    ze = jnp.zeros((E_TAB, D), _f32)
    zv = jnp.zeros((V_TAB, D), _f32)
    ecnt = _v2e(jnp.ones((NV, D), _f32), vg, es, ze)
    vcnt = _e2v(jnp.ones((NE, D), _f32), eg, vs, zv)

    Wg = [(W_g0, b_g0), (W_g1, b_g1), (W_g2, b_g2)]
    X = _mm0(x, W_g0, b_g0)
    for i in range(3):
        ep = _v2e(X, vg, es, ze)
        ef = _combe(ep, ecnt)
        qp = _e2v(ef, eg, vs, zv)
        if i < 2:
            X = _cvm(qp, vcnt, topos[i:i + 1], Wg[i + 1][0], Wg[i + 1][1])
        else:
            xf = _final(qp, vcnt, topos[2:3])
    return (xf, topos[2:3])
